# Initial kernel scaffold; baseline (speedup 1.0000x reference)
import jax
import jax.numpy as jnp
from jax.experimental import pallas as pl


def kernel(x, edge_index, pos_edge_index, neg_edge_index, W1, b1, W2, b2):
    raise NotImplementedError("write your pallas kernel here")


# speedup vs baseline: 2.1213x; 2.1213x over previous
import functools

import jax
import jax.numpy as jnp
from jax import lax
from jax.experimental import pallas as pl
from jax.experimental.pallas import tpu as pltpu
from jax.experimental.pallas import tpu_sc as plsc

N_NODES = 10000
D = 128
NC, NS, L = 2, 16, 16
NW = NC * NS
NPAD = 10240
CH = 128
K_AGG = 82
E_AGG = NW * CH * K_AGG
K_DEC = 80
E_DEC = NW * CH * K_DEC
DUMMY = N_NODES
ROWS_PER_TILE = NPAD // NS

_MESH = plsc.VectorSubcoreMesh(
    core_axis_name="c", subcore_axis_name="s", num_cores=NC, num_subcores=NS)
_SC_PARAMS = pltpu.CompilerParams(needs_layout_passes=False)
_SC_PARAMS_NT = pltpu.CompilerParams(
    needs_layout_passes=False, use_tc_tiling_on_sc=False)


@functools.partial(
    pl.kernel,
    out_type=jax.ShapeDtypeStruct((NC * NPAD, 16), jnp.float32),
    mesh=_MESH,
    scratch_types=[
        pltpu.VMEM((K_AGG, CH), jnp.int32),
        pltpu.VMEM((CH, 16), jnp.float32),
        pltpu.VMEM_SHARED((NPAD, 16), jnp.float32),
    ],
    compiler_params=_SC_PARAMS_NT,
)
def _deg_kernel(dst_hbm, zeros_hbm, out_hbm, dst_v, ones_v, acc_sh):
    c = lax.axis_index("c")
    s = lax.axis_index("s")
    wid = c * NS + s

    @pl.when(s == 0)
    def _():
        pltpu.sync_copy(zeros_hbm, acc_sh)

    for i in range(CH):
        ones_v[i] = jnp.full((16,), 1.0, jnp.float32)
    pltpu.sync_copy(dst_hbm.at[wid], dst_v)
    plsc.subcore_barrier()

    def chunk(k, carry):
        pltpu.sync_copy(ones_v, acc_sh.at[dst_v.at[k]], add=True)
        return carry

    lax.fori_loop(0, K_AGG, chunk, 0)
    plsc.subcore_barrier()
    pltpu.sync_copy(
        acc_sh.at[pl.ds(s * ROWS_PER_TILE, ROWS_PER_TILE)],
        out_hbm.at[pl.ds(c * NPAD + s * ROWS_PER_TILE, ROWS_PER_TILE)])


@functools.partial(
    pl.kernel,
    out_type=jax.ShapeDtypeStruct((NC * NPAD, D), jnp.float32),
    mesh=_MESH,
    scratch_types=[
        pltpu.VMEM((K_AGG, CH), jnp.int32),
        pltpu.VMEM((K_AGG, CH), jnp.int32),
        pltpu.VMEM((CH, D), jnp.float32),
        pltpu.VMEM_SHARED((NPAD, D), jnp.float32),
        pltpu.SemaphoreType.DMA,
    ],
    compiler_params=_SC_PARAMS,
)
def _agg_kernel(table_hbm, src_hbm, dst_hbm, zeros_hbm, out_hbm,
                src_v, dst_v, rows_v, acc_sh, sem):
    c = lax.axis_index("c")
    s = lax.axis_index("s")
    wid = c * NS + s

    @pl.when(s == 0)
    def _():
        pltpu.sync_copy(zeros_hbm, acc_sh)

    pltpu.sync_copy(src_hbm.at[wid], src_v)
    pltpu.sync_copy(dst_hbm.at[wid], dst_v)
    plsc.subcore_barrier()

    def chunk(k, carry):
        pltpu.async_copy(table_hbm.at[src_v.at[k]], rows_v, sem).wait()
        pltpu.sync_copy(rows_v, acc_sh.at[dst_v.at[k]], add=True)
        return carry

    lax.fori_loop(0, K_AGG, chunk, 0)
    plsc.subcore_barrier()
    pltpu.sync_copy(
        acc_sh.at[pl.ds(s * ROWS_PER_TILE, ROWS_PER_TILE)],
        out_hbm.at[pl.ds(c * NPAD + s * ROWS_PER_TILE, ROWS_PER_TILE)])


@functools.partial(
    pl.kernel,
    out_type=jax.ShapeDtypeStruct((NW * K_DEC, CH), jnp.float32),
    mesh=_MESH,
    scratch_types=[
        pltpu.VMEM((K_DEC, CH), jnp.int32),
        pltpu.VMEM((K_DEC, CH), jnp.int32),
        pltpu.VMEM((CH, D), jnp.float32),
        pltpu.VMEM((CH, D), jnp.float32),
        pltpu.VMEM((K_DEC, CH), jnp.float32),
        pltpu.SemaphoreType.DMA,
    ],
    compiler_params=_SC_PARAMS,
)
def _dec_kernel(z_hbm, ia_hbm, ib_hbm, out_hbm, ia_v, ib_v, ra_v, rb_v, sc_v, sem):
    c = lax.axis_index("c")
    s = lax.axis_index("s")
    wid = c * NS + s
    pltpu.sync_copy(ia_hbm.at[wid], ia_v)
    pltpu.sync_copy(ib_hbm.at[wid], ib_v)
    lane = lax.iota(jnp.int32, 16)

    def chunk(k, carry):
        pltpu.async_copy(z_hbm.at[ia_v.at[k]], ra_v, sem).wait()
        pltpu.async_copy(z_hbm.at[ib_v.at[k]], rb_v, sem).wait()

        def group(g, carry2):
            e_idx = g * L + lane
            score = jnp.zeros((L,), jnp.float32)
            for f in range(D):
                f_idx = jnp.full((L,), f, jnp.int32)
                av = plsc.load_gather(ra_v, [e_idx, f_idx])
                bv = plsc.load_gather(rb_v, [e_idx, f_idx])
                score = score + av * bv
            sc_v[k, pl.ds(g * L, L)] = score
            return carry2

        lax.fori_loop(0, CH // L, group, 0)
        return carry

    lax.fori_loop(0, K_DEC, chunk, 0)
    pltpu.sync_copy(sc_v, out_hbm.at[pl.ds(wid * K_DEC, K_DEC)])


_R = 1280


def _dinv_of(d0, d1):
    deg = d0[:, 0:1] + d1[:, 0:1]
    return jnp.where(deg > 0, lax.rsqrt(deg), 0.0)


def _tc1_body(x_ref, w_ref, d0_ref, d1_ref, o_ref):
    dinv = _dinv_of(d0_ref[...], d1_ref[...])
    o_ref[...] = jnp.dot(x_ref[...], w_ref[...],
                         preferred_element_type=jnp.float32) * dinv


def _tc2_body(p0_ref, p1_ref, d0_ref, d1_ref, b_ref, w_ref, o_ref):
    dinv = _dinv_of(d0_ref[...], d1_ref[...])
    h = jnp.maximum((p0_ref[...] + p1_ref[...]) * dinv + b_ref[...], 0.0)
    o_ref[...] = jnp.dot(h, w_ref[...], preferred_element_type=jnp.float32) * dinv


def _tc3_body(p0_ref, p1_ref, d0_ref, d1_ref, b_ref, o_ref):
    dinv = _dinv_of(d0_ref[...], d1_ref[...])
    o_ref[...] = (p0_ref[...] + p1_ref[...]) * dinv + b_ref[...]


def _row_spec(cols):
    return pl.BlockSpec((_R, cols), lambda i: (i, 0))


def _full_spec(rows, cols):
    return pl.BlockSpec((rows, cols), lambda i: (0, 0))


_tc1 = pl.pallas_call(
    _tc1_body,
    grid=(NPAD // _R,),
    in_specs=[_row_spec(D), _full_spec(D, D), _row_spec(16), _row_spec(16)],
    out_specs=_row_spec(D),
    out_shape=jax.ShapeDtypeStruct((NPAD, D), jnp.float32),
)

_tc2 = pl.pallas_call(
    _tc2_body,
    grid=(NPAD // _R,),
    in_specs=[_row_spec(D), _row_spec(D), _row_spec(16), _row_spec(16),
              _full_spec(1, D), _full_spec(D, D)],
    out_specs=_row_spec(D),
    out_shape=jax.ShapeDtypeStruct((NPAD, D), jnp.float32),
)

_tc3 = pl.pallas_call(
    _tc3_body,
    grid=(NPAD // _R,),
    in_specs=[_row_spec(D), _row_spec(D), _row_spec(16), _row_spec(16),
              _full_spec(1, D)],
    out_specs=_row_spec(D),
    out_shape=jax.ShapeDtypeStruct((NPAD, D), jnp.float32),
)


def kernel(x, edge_index, pos_edge_index, neg_edge_index, W1, b1, W2, b2):
    f32 = jnp.float32
    i32 = jnp.int32
    n_edges = edge_index.shape[1]

    src = edge_index[0].astype(i32)
    dst = edge_index[1].astype(i32)
    loop = jnp.arange(N_NODES, dtype=i32)
    pad_a = jnp.full((E_AGG - n_edges - N_NODES,), DUMMY, i32)
    src_sl = jnp.concatenate([src, loop, pad_a]).reshape(NW, K_AGG, CH)
    dst_sl = jnp.concatenate([dst, loop, pad_a]).reshape(NW, K_AGG, CH)

    zeros_d = jnp.zeros((NPAD, D), f32)
    zeros_16 = jnp.zeros((NPAD, 16), f32)
    x_pad = jnp.concatenate([x, jnp.zeros((NPAD - N_NODES, D), f32)], axis=0)

    degp = _deg_kernel(dst_sl, zeros_16)
    deg0, deg1 = degp[:NPAD], degp[NPAD:]

    xw1 = _tc1(x_pad, W1, deg0, deg1)
    acc1 = _agg_kernel(xw1, src_sl, dst_sl, zeros_d)
    xw2 = _tc2(acc1[:NPAD], acc1[NPAD:], deg0, deg1, b1.reshape(1, D), W2)
    acc2 = _agg_kernel(xw2, src_sl, dst_sl, zeros_d)
    z_pad = _tc3(acc2[:NPAD], acc2[NPAD:], deg0, deg1, b2.reshape(1, D))

    n_dec = pos_edge_index.shape[1]
    pad_d = jnp.full((E_DEC - n_dec,), DUMMY, i32)

    def _dec_idx(arr):
        return jnp.concatenate([arr.astype(i32), pad_d]).reshape(NW, K_DEC, CH)

    pos = _dec_kernel(z_pad, _dec_idx(pos_edge_index[0]),
                      _dec_idx(pos_edge_index[1])).reshape(-1)[:n_dec]
    neg = _dec_kernel(z_pad, _dec_idx(neg_edge_index[0]),
                      _dec_idx(neg_edge_index[1])).reshape(-1)[:n_dec]
    return pos, neg, z_pad[:N_NODES]


# speedup vs baseline: 3.7085x; 1.7482x over previous
import functools

import jax
import jax.numpy as jnp
from jax import lax
from jax.experimental import pallas as pl
from jax.experimental.pallas import tpu as pltpu
from jax.experimental.pallas import tpu_sc as plsc

N_NODES = 10000
D = 128
NC, NS, L = 2, 16, 16
NW = NC * NS
NPAD = 10240
CH = 128
K_AGG = 82
E_AGG = NW * CH * K_AGG
K_DEC = 80
E_DEC = NW * CH * K_DEC
DUMMY = N_NODES
ROWS_PER_TILE = NPAD // NS

_MESH = plsc.VectorSubcoreMesh(
    core_axis_name="c", subcore_axis_name="s", num_cores=NC, num_subcores=NS)
_SC_PARAMS = pltpu.CompilerParams(needs_layout_passes=False)
_SC_PARAMS_NT = pltpu.CompilerParams(
    needs_layout_passes=False, use_tc_tiling_on_sc=False)


@functools.partial(
    pl.kernel,
    out_type=jax.ShapeDtypeStruct((NC * NPAD, 16), jnp.float32),
    mesh=_MESH,
    scratch_types=[
        pltpu.VMEM((K_AGG, CH), jnp.int32),
        pltpu.VMEM((CH, 16), jnp.float32),
        pltpu.VMEM_SHARED((NPAD, 16), jnp.float32),
    ],
    compiler_params=_SC_PARAMS_NT,
)
def _deg_kernel(dst_hbm, zeros_hbm, out_hbm, dst_v, ones_v, acc_sh):
    c = lax.axis_index("c")
    s = lax.axis_index("s")
    wid = c * NS + s

    @pl.when(s == 0)
    def _():
        pltpu.sync_copy(zeros_hbm, acc_sh)

    for i in range(CH):
        ones_v[i] = jnp.full((16,), 1.0, jnp.float32)
    pltpu.sync_copy(dst_hbm.at[wid], dst_v)
    plsc.subcore_barrier()

    def chunk(k, carry):
        pltpu.sync_copy(ones_v, acc_sh.at[dst_v.at[k]], add=True)
        return carry

    lax.fori_loop(0, K_AGG, chunk, 0)
    plsc.subcore_barrier()
    pltpu.sync_copy(
        acc_sh.at[pl.ds(s * ROWS_PER_TILE, ROWS_PER_TILE)],
        out_hbm.at[pl.ds(c * NPAD + s * ROWS_PER_TILE, ROWS_PER_TILE)])


@functools.partial(
    pl.kernel,
    out_type=jax.ShapeDtypeStruct((NC * NPAD, D), jnp.float32),
    mesh=_MESH,
    scratch_types=[
        pltpu.VMEM((K_AGG, CH), jnp.int32),
        pltpu.VMEM((K_AGG, CH), jnp.int32),
        pltpu.VMEM((CH, D), jnp.float32),
        pltpu.VMEM_SHARED((NPAD, D), jnp.float32),
        pltpu.SemaphoreType.DMA,
    ],
    compiler_params=_SC_PARAMS,
)
def _agg_kernel(table_hbm, src_hbm, dst_hbm, zeros_hbm, out_hbm,
                src_v, dst_v, rows_v, acc_sh, sem):
    c = lax.axis_index("c")
    s = lax.axis_index("s")
    wid = c * NS + s

    @pl.when(s == 0)
    def _():
        pltpu.sync_copy(zeros_hbm, acc_sh)

    pltpu.sync_copy(src_hbm.at[wid], src_v)
    pltpu.sync_copy(dst_hbm.at[wid], dst_v)
    plsc.subcore_barrier()

    def chunk(k, carry):
        pltpu.async_copy(table_hbm.at[src_v.at[k]], rows_v, sem).wait()
        pltpu.sync_copy(rows_v, acc_sh.at[dst_v.at[k]], add=True)
        return carry

    lax.fori_loop(0, K_AGG, chunk, 0)
    plsc.subcore_barrier()
    pltpu.sync_copy(
        acc_sh.at[pl.ds(s * ROWS_PER_TILE, ROWS_PER_TILE)],
        out_hbm.at[pl.ds(c * NPAD + s * ROWS_PER_TILE, ROWS_PER_TILE)])


@functools.partial(
    pl.kernel,
    out_type=jax.ShapeDtypeStruct((NW * K_DEC, CH), jnp.float32),
    mesh=_MESH,
    scratch_types=[
        pltpu.VMEM((K_DEC, CH), jnp.int32),
        pltpu.VMEM((K_DEC, CH), jnp.int32),
        pltpu.VMEM((2, CH, D), jnp.float32),
        pltpu.VMEM((2, CH, D), jnp.float32),
        pltpu.VMEM((L, L), jnp.float32),
        pltpu.VMEM((K_DEC, CH), jnp.float32),
        pltpu.SemaphoreType.DMA((2,)),
    ],
    compiler_params=_SC_PARAMS,
)
def _dec_kernel(z_hbm, ia_hbm, ib_hbm, out_hbm, ia_v, ib_v, ra_v, rb_v, pt_v,
                sc_v, sems):
    c = lax.axis_index("c")
    s = lax.axis_index("s")
    wid = c * NS + s
    pltpu.sync_copy(ia_hbm.at[wid], ia_v)
    pltpu.sync_copy(ib_hbm.at[wid], ib_v)
    lane = lax.iota(jnp.int32, 16)

    def _issue(k, b):
        pltpu.async_copy(z_hbm.at[ia_v.at[k]], ra_v.at[b], sems.at[b])
        pltpu.async_copy(z_hbm.at[ib_v.at[k]], rb_v.at[b], sems.at[b])

    def _drain(k, b):
        pltpu.make_async_copy(z_hbm.at[ia_v.at[k]], ra_v.at[b], sems.at[b]).wait()
        pltpu.make_async_copy(z_hbm.at[ib_v.at[k]], rb_v.at[b], sems.at[b]).wait()

    def _compute(k, b):
        ra = ra_v.at[b]
        rb = rb_v.at[b]

        def group(g, carry2):
            for e in range(L):
                row = g * L + e
                ps = [ra[row, pl.ds(j * L, L)] * rb[row, pl.ds(j * L, L)]
                      for j in range(D // L)]
                part = (((ps[0] + ps[1]) + (ps[2] + ps[3]))
                        + ((ps[4] + ps[5]) + (ps[6] + ps[7])))
                pt_v[e] = part
            score = plsc.load_gather(pt_v, [lane, jnp.zeros((L,), jnp.int32)])
            for j in range(1, L):
                score = score + plsc.load_gather(
                    pt_v, [lane, jnp.full((L,), j, jnp.int32)])
            sc_v[k, pl.ds(g * L, L)] = score
            return carry2

        lax.fori_loop(0, CH // L, group, 0)

    _issue(0, 0)
    _issue(1, 1)

    def pair(g, carry):
        for b in range(2):
            k = 2 * g + b
            _drain(k, b)
            _compute(k, b)

            @pl.when(k + 2 < K_DEC)
            def _():
                _issue(k + 2, b)
        return carry

    lax.fori_loop(0, K_DEC // 2, pair, 0)
    pltpu.sync_copy(sc_v, out_hbm.at[pl.ds(wid * K_DEC, K_DEC)])


_R = 1280


def _dinv_of(d0, d1):
    deg = d0[:, 0:1] + d1[:, 0:1]
    return jnp.where(deg > 0, lax.rsqrt(deg), 0.0)


def _tc1_body(x_ref, w_ref, d0_ref, d1_ref, o_ref):
    dinv = _dinv_of(d0_ref[...], d1_ref[...])
    o_ref[...] = jnp.dot(x_ref[...], w_ref[...],
                         preferred_element_type=jnp.float32) * dinv


def _tc2_body(p0_ref, p1_ref, d0_ref, d1_ref, b_ref, w_ref, o_ref):
    dinv = _dinv_of(d0_ref[...], d1_ref[...])
    h = jnp.maximum((p0_ref[...] + p1_ref[...]) * dinv + b_ref[...], 0.0)
    o_ref[...] = jnp.dot(h, w_ref[...], preferred_element_type=jnp.float32) * dinv


def _tc3_body(p0_ref, p1_ref, d0_ref, d1_ref, b_ref, o_ref):
    dinv = _dinv_of(d0_ref[...], d1_ref[...])
    o_ref[...] = (p0_ref[...] + p1_ref[...]) * dinv + b_ref[...]


def _row_spec(cols):
    return pl.BlockSpec((_R, cols), lambda i: (i, 0))


def _full_spec(rows, cols):
    return pl.BlockSpec((rows, cols), lambda i: (0, 0))


_tc1 = pl.pallas_call(
    _tc1_body,
    grid=(NPAD // _R,),
    in_specs=[_row_spec(D), _full_spec(D, D), _row_spec(16), _row_spec(16)],
    out_specs=_row_spec(D),
    out_shape=jax.ShapeDtypeStruct((NPAD, D), jnp.float32),
)

_tc2 = pl.pallas_call(
    _tc2_body,
    grid=(NPAD // _R,),
    in_specs=[_row_spec(D), _row_spec(D), _row_spec(16), _row_spec(16),
              _full_spec(1, D), _full_spec(D, D)],
    out_specs=_row_spec(D),
    out_shape=jax.ShapeDtypeStruct((NPAD, D), jnp.float32),
)

_tc3 = pl.pallas_call(
    _tc3_body,
    grid=(NPAD // _R,),
    in_specs=[_row_spec(D), _row_spec(D), _row_spec(16), _row_spec(16),
              _full_spec(1, D)],
    out_specs=_row_spec(D),
    out_shape=jax.ShapeDtypeStruct((NPAD, D), jnp.float32),
)


def kernel(x, edge_index, pos_edge_index, neg_edge_index, W1, b1, W2, b2):
    f32 = jnp.float32
    i32 = jnp.int32
    n_edges = edge_index.shape[1]

    src = edge_index[0].astype(i32)
    dst = edge_index[1].astype(i32)
    loop = jnp.arange(N_NODES, dtype=i32)
    pad_a = jnp.full((E_AGG - n_edges - N_NODES,), DUMMY, i32)
    src_sl = jnp.concatenate([src, loop, pad_a]).reshape(NW, K_AGG, CH)
    dst_sl = jnp.concatenate([dst, loop, pad_a]).reshape(NW, K_AGG, CH)

    zeros_d = jnp.zeros((NPAD, D), f32)
    zeros_16 = jnp.zeros((NPAD, 16), f32)
    x_pad = jnp.concatenate([x, jnp.zeros((NPAD - N_NODES, D), f32)], axis=0)

    degp = _deg_kernel(dst_sl, zeros_16)
    deg0, deg1 = degp[:NPAD], degp[NPAD:]

    xw1 = _tc1(x_pad, W1, deg0, deg1)
    acc1 = _agg_kernel(xw1, src_sl, dst_sl, zeros_d)
    xw2 = _tc2(acc1[:NPAD], acc1[NPAD:], deg0, deg1, b1.reshape(1, D), W2)
    acc2 = _agg_kernel(xw2, src_sl, dst_sl, zeros_d)
    z_pad = _tc3(acc2[:NPAD], acc2[NPAD:], deg0, deg1, b2.reshape(1, D))

    n_dec = pos_edge_index.shape[1]
    pad_d = jnp.full((E_DEC - n_dec,), DUMMY, i32)

    def _dec_idx(arr):
        return jnp.concatenate([arr.astype(i32), pad_d]).reshape(NW, K_DEC, CH)

    pos = _dec_kernel(z_pad, _dec_idx(pos_edge_index[0]),
                      _dec_idx(pos_edge_index[1])).reshape(-1)[:n_dec]
    neg = _dec_kernel(z_pad, _dec_idx(neg_edge_index[0]),
                      _dec_idx(neg_edge_index[1])).reshape(-1)[:n_dec]
    return pos, neg, z_pad[:N_NODES]


# speedup vs baseline: 3.7204x; 1.0032x over previous
import functools

import jax
import jax.numpy as jnp
from jax import lax
from jax.experimental import pallas as pl
from jax.experimental.pallas import tpu as pltpu
from jax.experimental.pallas import tpu_sc as plsc

N_NODES = 10000
D = 128
NC, NS, L = 2, 16, 16
NW = NC * NS
NPAD = 10240
CH = 128
K_AGG = 82
E_AGG = NW * CH * K_AGG
K_DEC = 80
E_DEC = NW * CH * K_DEC
DUMMY = N_NODES
ROWS_PER_TILE = NPAD // NS

_MESH = plsc.VectorSubcoreMesh(
    core_axis_name="c", subcore_axis_name="s", num_cores=NC, num_subcores=NS)
_SC_PARAMS = pltpu.CompilerParams(needs_layout_passes=False)
_SC_PARAMS_NT = pltpu.CompilerParams(
    needs_layout_passes=False, use_tc_tiling_on_sc=False)


@functools.partial(
    pl.kernel,
    out_type=jax.ShapeDtypeStruct((NC * NPAD, 16), jnp.float32),
    mesh=_MESH,
    scratch_types=[
        pltpu.VMEM((K_AGG, CH), jnp.int32),
        pltpu.VMEM((CH, 16), jnp.float32),
        pltpu.VMEM_SHARED((NPAD, 16), jnp.float32),
    ],
    compiler_params=_SC_PARAMS_NT,
)
def _deg_kernel(dst_hbm, zeros_hbm, out_hbm, dst_v, ones_v, acc_sh):
    c = lax.axis_index("c")
    s = lax.axis_index("s")
    wid = c * NS + s

    @pl.when(s == 0)
    def _():
        pltpu.sync_copy(zeros_hbm, acc_sh)

    for i in range(CH):
        ones_v[i] = jnp.full((16,), 1.0, jnp.float32)
    pltpu.sync_copy(dst_hbm.at[wid], dst_v)
    plsc.subcore_barrier()

    def chunk(k, carry):
        pltpu.sync_copy(ones_v, acc_sh.at[dst_v.at[k]], add=True)
        return carry

    lax.fori_loop(0, K_AGG, chunk, 0)
    plsc.subcore_barrier()
    pltpu.sync_copy(
        acc_sh.at[pl.ds(s * ROWS_PER_TILE, ROWS_PER_TILE)],
        out_hbm.at[pl.ds(c * NPAD + s * ROWS_PER_TILE, ROWS_PER_TILE)])


@functools.partial(
    pl.kernel,
    out_type=jax.ShapeDtypeStruct((NC * NPAD, D), jnp.float32),
    mesh=_MESH,
    scratch_types=[
        pltpu.VMEM((K_AGG, CH), jnp.int32),
        pltpu.VMEM((K_AGG, CH), jnp.int32),
        pltpu.VMEM((CH, D), jnp.float32),
        pltpu.VMEM_SHARED((NPAD, D), jnp.float32),
        pltpu.SemaphoreType.DMA,
    ],
    compiler_params=_SC_PARAMS,
)
def _agg_kernel(table_hbm, src_hbm, dst_hbm, zeros_hbm, out_hbm,
                src_v, dst_v, rows_v, acc_sh, sem):
    c = lax.axis_index("c")
    s = lax.axis_index("s")
    wid = c * NS + s

    @pl.when(s == 0)
    def _():
        pltpu.sync_copy(zeros_hbm, acc_sh)

    pltpu.sync_copy(src_hbm.at[wid], src_v)
    pltpu.sync_copy(dst_hbm.at[wid], dst_v)
    plsc.subcore_barrier()

    def chunk(k, carry):
        pltpu.async_copy(table_hbm.at[src_v.at[k]], rows_v, sem).wait()
        pltpu.sync_copy(rows_v, acc_sh.at[dst_v.at[k]], add=True)
        return carry

    lax.fori_loop(0, K_AGG, chunk, 0)
    plsc.subcore_barrier()
    pltpu.sync_copy(
        acc_sh.at[pl.ds(s * ROWS_PER_TILE, ROWS_PER_TILE)],
        out_hbm.at[pl.ds(c * NPAD + s * ROWS_PER_TILE, ROWS_PER_TILE)])


@functools.partial(
    pl.kernel,
    out_type=jax.ShapeDtypeStruct((NW * K_DEC, CH), jnp.float32),
    mesh=_MESH,
    scratch_types=[
        pltpu.VMEM((K_DEC, CH), jnp.int32),
        pltpu.VMEM((K_DEC, CH), jnp.int32),
        pltpu.VMEM((2, CH, D), jnp.float32),
        pltpu.VMEM((2, CH, D), jnp.float32),
        pltpu.VMEM((L, L), jnp.float32),
        pltpu.VMEM((K_DEC, CH), jnp.float32),
        pltpu.SemaphoreType.DMA((2,)),
    ],
    compiler_params=_SC_PARAMS,
)
def _dec_kernel(z_hbm, ia_hbm, ib_hbm, out_hbm, ia_v, ib_v, ra_v, rb_v, pt_v,
                sc_v, sems):
    c = lax.axis_index("c")
    s = lax.axis_index("s")
    wid = c * NS + s
    z_c = z_hbm.at[c]
    pltpu.sync_copy(ia_hbm.at[wid], ia_v)
    pltpu.sync_copy(ib_hbm.at[wid], ib_v)
    lane = lax.iota(jnp.int32, 16)

    def _issue(k, b):
        pltpu.async_copy(z_c.at[ia_v.at[k]], ra_v.at[b], sems.at[b])
        pltpu.async_copy(z_c.at[ib_v.at[k]], rb_v.at[b], sems.at[b])

    def _drain(k, b):
        pltpu.make_async_copy(z_c.at[ia_v.at[k]], ra_v.at[b], sems.at[b]).wait()
        pltpu.make_async_copy(z_c.at[ib_v.at[k]], rb_v.at[b], sems.at[b]).wait()

    def _compute(k, b):
        ra = ra_v.at[b]
        rb = rb_v.at[b]

        def group(g, carry2):
            for e in range(L):
                row = g * L + e
                ps = [ra[row, pl.ds(j * L, L)] * rb[row, pl.ds(j * L, L)]
                      for j in range(D // L)]
                part = (((ps[0] + ps[1]) + (ps[2] + ps[3]))
                        + ((ps[4] + ps[5]) + (ps[6] + ps[7])))
                pt_v[e] = part
            score = plsc.load_gather(pt_v, [lane, jnp.zeros((L,), jnp.int32)])
            for j in range(1, L):
                score = score + plsc.load_gather(
                    pt_v, [lane, jnp.full((L,), j, jnp.int32)])
            sc_v[k, pl.ds(g * L, L)] = score
            return carry2

        lax.fori_loop(0, CH // L, group, 0)

    _issue(0, 0)
    _issue(1, 1)

    def pair(g, carry):
        for b in range(2):
            k = 2 * g + b
            _drain(k, b)
            _compute(k, b)

            @pl.when(k + 2 < K_DEC)
            def _():
                _issue(k + 2, b)
        return carry

    lax.fori_loop(0, K_DEC // 2, pair, 0)
    pltpu.sync_copy(sc_v, out_hbm.at[pl.ds(wid * K_DEC, K_DEC)])


_R = 1280


def _dinv_of(d0, d1):
    deg = d0[:, 0:1] + d1[:, 0:1]
    return jnp.where(deg > 0, lax.rsqrt(deg), 0.0)


def _tc1_body(x_ref, w_ref, d0_ref, d1_ref, o_ref):
    dinv = _dinv_of(d0_ref[...], d1_ref[...])
    o_ref[...] = jnp.dot(x_ref[...], w_ref[...],
                         preferred_element_type=jnp.float32) * dinv


def _tc2_body(p0_ref, p1_ref, d0_ref, d1_ref, b_ref, w_ref, o_ref):
    dinv = _dinv_of(d0_ref[...], d1_ref[...])
    h = jnp.maximum((p0_ref[...] + p1_ref[...]) * dinv + b_ref[...], 0.0)
    o_ref[...] = jnp.dot(h, w_ref[...], preferred_element_type=jnp.float32) * dinv


def _tc3_body(p0_ref, p1_ref, d0_ref, d1_ref, b_ref, o_ref):
    dinv = _dinv_of(d0_ref[...], d1_ref[...])
    o_ref[...] = (p0_ref[...] + p1_ref[...]) * dinv + b_ref[...]


def _row_spec(cols):
    return pl.BlockSpec((_R, cols), lambda i: (i, 0))


def _full_spec(rows, cols):
    return pl.BlockSpec((rows, cols), lambda i: (0, 0))


_tc1 = pl.pallas_call(
    _tc1_body,
    grid=(NPAD // _R,),
    in_specs=[_row_spec(D), _full_spec(D, D), _row_spec(16), _row_spec(16)],
    out_specs=_row_spec(D),
    out_shape=jax.ShapeDtypeStruct((NPAD, D), jnp.float32),
)

_tc2 = pl.pallas_call(
    _tc2_body,
    grid=(NPAD // _R,),
    in_specs=[_row_spec(D), _row_spec(D), _row_spec(16), _row_spec(16),
              _full_spec(1, D), _full_spec(D, D)],
    out_specs=_row_spec(D),
    out_shape=jax.ShapeDtypeStruct((NPAD, D), jnp.float32),
)

_tc3 = pl.pallas_call(
    _tc3_body,
    grid=(NPAD // _R,),
    in_specs=[_row_spec(D), _row_spec(D), _row_spec(16), _row_spec(16),
              _full_spec(1, D)],
    out_specs=_row_spec(D),
    out_shape=jax.ShapeDtypeStruct((NPAD, D), jnp.float32),
)


def kernel(x, edge_index, pos_edge_index, neg_edge_index, W1, b1, W2, b2):
    f32 = jnp.float32
    i32 = jnp.int32
    n_edges = edge_index.shape[1]

    src = edge_index[0].astype(i32)
    dst = edge_index[1].astype(i32)
    loop = jnp.arange(N_NODES, dtype=i32)
    pad_a = jnp.full((E_AGG - n_edges - N_NODES,), DUMMY, i32)
    src_sl = jnp.concatenate([src, loop, pad_a]).reshape(NW, K_AGG, CH)
    dst_sl = jnp.concatenate([dst, loop, pad_a]).reshape(NW, K_AGG, CH)

    zeros_d = jnp.zeros((NPAD, D), f32)
    zeros_16 = jnp.zeros((NPAD, 16), f32)
    x_pad = jnp.concatenate([x, jnp.zeros((NPAD - N_NODES, D), f32)], axis=0)

    degp = _deg_kernel(dst_sl, zeros_16)
    deg0, deg1 = degp[:NPAD], degp[NPAD:]

    xw1 = _tc1(x_pad, W1, deg0, deg1)
    acc1 = _agg_kernel(xw1, src_sl, dst_sl, zeros_d)
    xw2 = _tc2(acc1[:NPAD], acc1[NPAD:], deg0, deg1, b1.reshape(1, D), W2)
    acc2 = _agg_kernel(xw2, src_sl, dst_sl, zeros_d)
    z_pad = _tc3(acc2[:NPAD], acc2[NPAD:], deg0, deg1, b2.reshape(1, D))

    n_dec = pos_edge_index.shape[1]
    pad_d = jnp.full((E_DEC - n_dec,), DUMMY, i32)

    def _dec_idx(arr):
        return jnp.concatenate([arr.astype(i32), pad_d]).reshape(NW, K_DEC, CH)

    z_pair = jnp.stack([z_pad, z_pad])
    pos = _dec_kernel(z_pair, _dec_idx(pos_edge_index[0]),
                      _dec_idx(pos_edge_index[1])).reshape(-1)[:n_dec]
    neg = _dec_kernel(z_pair, _dec_idx(neg_edge_index[0]),
                      _dec_idx(neg_edge_index[1])).reshape(-1)[:n_dec]
    return pos, neg, z_pad[:N_NODES]


# speedup vs baseline: 4.4663x; 1.2005x over previous
import functools

import jax
import jax.numpy as jnp
from jax import lax
from jax.experimental import pallas as pl
from jax.experimental.pallas import tpu as pltpu
from jax.experimental.pallas import tpu_sc as plsc

N_NODES = 10000
D = 128
NC, NS, L = 2, 16, 16
NW = NC * NS
NPAD = 10240
CH = 128
K_AGG = 84
E_AGG = NW * CH * K_AGG
K_DEC = 80
E_DEC = NW * CH * K_DEC
DUMMY = N_NODES
ROWS_PER_TILE = NPAD // NS

_MESH = plsc.VectorSubcoreMesh(
    core_axis_name="c", subcore_axis_name="s", num_cores=NC, num_subcores=NS)
_SC_PARAMS = pltpu.CompilerParams(needs_layout_passes=False)
_SC_PARAMS_NT = pltpu.CompilerParams(
    needs_layout_passes=False, use_tc_tiling_on_sc=False)


@functools.partial(
    pl.kernel,
    out_type=jax.ShapeDtypeStruct((NW, NPAD), jnp.float32),
    mesh=_MESH,
    scratch_types=[
        pltpu.VMEM((K_AGG, CH), jnp.int32),
        pltpu.VMEM((NPAD,), jnp.float32),
    ],
    compiler_params=_SC_PARAMS,
)
def _deg_kernel(dst_hbm, out_hbm, dst_v, deg_v):
    c = lax.axis_index("c")
    s = lax.axis_index("s")
    wid = c * NS + s
    pltpu.sync_copy(dst_hbm.at[wid], dst_v)
    zeros = jnp.zeros((L,), jnp.float32)
    ones = jnp.full((L,), 1.0, jnp.float32)

    def zinit(i, carry):
        deg_v[pl.ds(i * L, L)] = zeros
        return carry

    lax.fori_loop(0, NPAD // L, zinit, 0)

    def chunk(k, carry):
        for j in range(CH // L):
            idx = dst_v[k, pl.ds(j * L, L)]
            plsc.addupdate_scatter(deg_v, [idx], ones)
        return carry

    lax.fori_loop(0, K_AGG, chunk, 0)
    pltpu.sync_copy(deg_v, out_hbm.at[wid])


@functools.partial(
    pl.kernel,
    out_type=jax.ShapeDtypeStruct((NC * NPAD, D), jnp.float32),
    mesh=_MESH,
    scratch_types=[
        pltpu.VMEM((K_AGG, CH), jnp.int32),
        pltpu.VMEM((K_AGG, CH), jnp.int32),
        pltpu.VMEM((CH, D), jnp.float32),
        pltpu.VMEM_SHARED((NPAD, D), jnp.float32),
        pltpu.SemaphoreType.DMA,
    ],
    compiler_params=_SC_PARAMS,
)
def _agg_kernel(table_hbm, src_hbm, dst_hbm, zeros_hbm, out_hbm,
                src_v, dst_v, rows_v, acc_sh, sem):
    c = lax.axis_index("c")
    s = lax.axis_index("s")
    wid = c * NS + s

    @pl.when(s == 0)
    def _():
        pltpu.sync_copy(zeros_hbm, acc_sh)

    pltpu.sync_copy(src_hbm.at[wid], src_v)
    pltpu.sync_copy(dst_hbm.at[wid], dst_v)
    plsc.subcore_barrier()

    def chunk(k, carry):
        pltpu.async_copy(table_hbm.at[src_v.at[k]], rows_v, sem).wait()
        pltpu.sync_copy(rows_v, acc_sh.at[dst_v.at[k]], add=True)
        return carry

    lax.fori_loop(0, K_AGG, chunk, 0)
    plsc.subcore_barrier()
    pltpu.sync_copy(
        acc_sh.at[pl.ds(s * ROWS_PER_TILE, ROWS_PER_TILE)],
        out_hbm.at[pl.ds(c * NPAD + s * ROWS_PER_TILE, ROWS_PER_TILE)])


@functools.partial(
    pl.kernel,
    out_type=jax.ShapeDtypeStruct((NW * K_DEC, CH), jnp.float32),
    mesh=_MESH,
    scratch_types=[
        pltpu.VMEM((K_DEC, CH), jnp.int32),
        pltpu.VMEM((K_DEC, CH), jnp.int32),
        pltpu.VMEM((4, CH, D // 2), jnp.float32),
        pltpu.VMEM((4, CH, D // 2), jnp.float32),
        pltpu.VMEM((L, L), jnp.float32),
        pltpu.VMEM((K_DEC, CH), jnp.float32),
        pltpu.SemaphoreType.DMA((4,)),
    ],
    compiler_params=_SC_PARAMS_NT,
)
def _dec_kernel(z_hbm, ia_hbm, ib_hbm, out_hbm, ia_v, ib_v, ra_v, rb_v, pt_v,
                sc_v, sems):
    c = lax.axis_index("c")
    s = lax.axis_index("s")
    wid = c * NS + s
    pltpu.sync_copy(ia_hbm.at[wid], ia_v)
    pltpu.sync_copy(ib_hbm.at[wid], ib_v)
    lane = lax.iota(jnp.int32, 16)

    def _issue(k, b):
        pltpu.async_copy(z_hbm.at[ia_v.at[k]], ra_v.at[b], sems.at[b])
        pltpu.async_copy(z_hbm.at[ib_v.at[k]], rb_v.at[b], sems.at[b])

    def _drain(k, b):
        pltpu.make_async_copy(z_hbm.at[ia_v.at[k]], ra_v.at[b], sems.at[b]).wait()
        pltpu.make_async_copy(z_hbm.at[ib_v.at[k]], rb_v.at[b], sems.at[b]).wait()

    def _compute(k, b):
        ra = ra_v.at[b]
        rb = rb_v.at[b]

        def group(g, carry2):
            for e in range(L):
                row = g * L + e
                ps = []
                for j in range(D // 32):
                    apk = plsc.bitcast(ra[row, pl.ds(j * L, L)], jnp.bfloat16)
                    bpk = plsc.bitcast(rb[row, pl.ds(j * L, L)], jnp.bfloat16)
                    a0, a1 = plsc.unpack(apk, format=plsc.PackFormat.INTERLEAVED)
                    b0, b1 = plsc.unpack(bpk, format=plsc.PackFormat.INTERLEAVED)
                    ps.append(a0 * b0)
                    ps.append(a1 * b1)
                part = (((ps[0] + ps[1]) + (ps[2] + ps[3]))
                        + ((ps[4] + ps[5]) + (ps[6] + ps[7])))
                pt_v[e] = part
            score = plsc.load_gather(pt_v, [lane, jnp.zeros((L,), jnp.int32)])
            for j in range(1, L):
                score = score + plsc.load_gather(
                    pt_v, [lane, jnp.full((L,), j, jnp.int32)])
            sc_v[k, pl.ds(g * L, L)] = score
            return carry2

        lax.fori_loop(0, CH // L, group, 0)

    for b in range(4):
        _issue(b, b)

    def ring(g, carry):
        for b in range(4):
            k = 4 * g + b
            _drain(k, b)
            _compute(k, b)

            @pl.when(k + 4 < K_DEC)
            def _():
                _issue(k + 4, b)
        return carry

    lax.fori_loop(0, K_DEC // 4, ring, 0)
    pltpu.sync_copy(sc_v, out_hbm.at[pl.ds(wid * K_DEC, K_DEC)])


_R = 1280


def _dinv_of(d_ref):
    deg = jnp.sum(d_ref[...], axis=0)[:, None]
    return jnp.where(deg > 0, lax.rsqrt(deg), 0.0)


def _tc1_body(x_ref, w_ref, d_ref, o_ref):
    dinv = _dinv_of(d_ref)
    o_ref[...] = jnp.dot(x_ref[...], w_ref[...],
                         preferred_element_type=jnp.float32) * dinv


def _tc2_body(p0_ref, p1_ref, d_ref, b_ref, w_ref, o_ref):
    dinv = _dinv_of(d_ref)
    h = jnp.maximum((p0_ref[...] + p1_ref[...]) * dinv + b_ref[...], 0.0)
    o_ref[...] = jnp.dot(h, w_ref[...], preferred_element_type=jnp.float32) * dinv


def _tc3_body(p0_ref, p1_ref, d_ref, b_ref, o_ref, ob_ref):
    dinv = _dinv_of(d_ref)
    z = (p0_ref[...] + p1_ref[...]) * dinv + b_ref[...]
    o_ref[...] = z
    ob_ref[...] = z.astype(jnp.bfloat16)


def _row_spec(cols):
    return pl.BlockSpec((_R, cols), lambda i: (i, 0))


def _full_spec(rows, cols):
    return pl.BlockSpec((rows, cols), lambda i: (0, 0))


_deg_spec = pl.BlockSpec((NW, _R), lambda i: (0, i))

_tc1 = pl.pallas_call(
    _tc1_body,
    grid=(NPAD // _R,),
    in_specs=[_row_spec(D), _full_spec(D, D), _deg_spec],
    out_specs=_row_spec(D),
    out_shape=jax.ShapeDtypeStruct((NPAD, D), jnp.float32),
)

_tc2 = pl.pallas_call(
    _tc2_body,
    grid=(NPAD // _R,),
    in_specs=[_row_spec(D), _row_spec(D), _deg_spec,
              _full_spec(1, D), _full_spec(D, D)],
    out_specs=_row_spec(D),
    out_shape=jax.ShapeDtypeStruct((NPAD, D), jnp.float32),
)

_tc3 = pl.pallas_call(
    _tc3_body,
    grid=(NPAD // _R,),
    in_specs=[_row_spec(D), _row_spec(D), _deg_spec, _full_spec(1, D)],
    out_specs=[_row_spec(D), _row_spec(D)],
    out_shape=[jax.ShapeDtypeStruct((NPAD, D), jnp.float32),
               jax.ShapeDtypeStruct((NPAD, D), jnp.bfloat16)],
)


def kernel(x, edge_index, pos_edge_index, neg_edge_index, W1, b1, W2, b2):
    f32 = jnp.float32
    i32 = jnp.int32
    n_edges = edge_index.shape[1]

    src = edge_index[0].astype(i32)
    dst = edge_index[1].astype(i32)
    loop = jnp.arange(N_NODES, dtype=i32)
    pad_a = jnp.full((E_AGG - n_edges - N_NODES,), DUMMY, i32)
    src_sl = jnp.concatenate([src, loop, pad_a]).reshape(NW, K_AGG, CH)
    dst_sl = jnp.concatenate([dst, loop, pad_a]).reshape(NW, K_AGG, CH)

    zeros_d = jnp.zeros((NPAD, D), f32)
    x_pad = jnp.concatenate([x, jnp.zeros((NPAD - N_NODES, D), f32)], axis=0)

    degs = _deg_kernel(dst_sl)

    xw1 = _tc1(x_pad, W1, degs)
    acc1 = _agg_kernel(xw1, src_sl, dst_sl, zeros_d)
    xw2 = _tc2(acc1[:NPAD], acc1[NPAD:], degs, b1.reshape(1, D), W2)
    acc2 = _agg_kernel(xw2, src_sl, dst_sl, zeros_d)
    z_pad, z_b16 = _tc3(acc2[:NPAD], acc2[NPAD:], degs, b2.reshape(1, D))
    z_pk = jax.lax.bitcast_convert_type(z_b16.reshape(NPAD, D // 2, 2),
                                        jnp.float32)

    n_dec = pos_edge_index.shape[1]
    pad_d = jnp.full((E_DEC - n_dec,), DUMMY, i32)

    def _dec_idx(arr):
        return jnp.concatenate([arr.astype(i32), pad_d]).reshape(NW, K_DEC, CH)

    pos = _dec_kernel(z_pk, _dec_idx(pos_edge_index[0]),
                      _dec_idx(pos_edge_index[1])).reshape(-1)[:n_dec]
    neg = _dec_kernel(z_pk, _dec_idx(neg_edge_index[0]),
                      _dec_idx(neg_edge_index[1])).reshape(-1)[:n_dec]
    return pos, neg, z_pad[:N_NODES]


# speedup vs baseline: 6.0954x; 1.3648x over previous
import functools

import jax
import jax.numpy as jnp
from jax import lax
from jax.experimental import pallas as pl
from jax.experimental.pallas import tpu as pltpu
from jax.experimental.pallas import tpu_sc as plsc

N_NODES = 10000
D = 128
NC, NS, L = 2, 16, 16
NW = NC * NS
NPAD = 10240
CH = 128
K_AGG = 84
E_AGG = NW * CH * K_AGG
K_DEC = 80
E_DEC = NW * CH * K_DEC
DUMMY = N_NODES
ROWS_PER_TILE = NPAD // NS

_MESH = plsc.VectorSubcoreMesh(
    core_axis_name="c", subcore_axis_name="s", num_cores=NC, num_subcores=NS)
_SC_PARAMS = pltpu.CompilerParams(needs_layout_passes=False)
_SC_PARAMS_NT = pltpu.CompilerParams(
    needs_layout_passes=False, use_tc_tiling_on_sc=False)


@functools.partial(
    pl.kernel,
    out_type=jax.ShapeDtypeStruct((NW, NPAD), jnp.float32),
    mesh=_MESH,
    scratch_types=[
        pltpu.VMEM((K_AGG, CH), jnp.int32),
        pltpu.VMEM((NPAD,), jnp.float32),
    ],
    compiler_params=_SC_PARAMS,
)
def _deg_kernel(dst_hbm, out_hbm, dst_v, deg_v):
    c = lax.axis_index("c")
    s = lax.axis_index("s")
    wid = c * NS + s
    pltpu.sync_copy(dst_hbm.at[wid], dst_v)
    zeros = jnp.zeros((L,), jnp.float32)
    ones = jnp.full((L,), 1.0, jnp.float32)

    def zinit(i, carry):
        deg_v[pl.ds(i * L, L)] = zeros
        return carry

    lax.fori_loop(0, NPAD // L, zinit, 0)

    def chunk(k, carry):
        for j in range(CH // L):
            idx = dst_v[k, pl.ds(j * L, L)]
            plsc.addupdate_scatter(deg_v, [idx], ones)
        return carry

    lax.fori_loop(0, K_AGG, chunk, 0)
    pltpu.sync_copy(deg_v, out_hbm.at[wid])


@functools.partial(
    pl.kernel,
    out_type=jax.ShapeDtypeStruct((NC * NPAD, D), jnp.float32),
    mesh=_MESH,
    scratch_types=[
        pltpu.VMEM((K_AGG, CH), jnp.int32),
        pltpu.VMEM((K_AGG, CH), jnp.int32),
        pltpu.VMEM((CH, D // 2), jnp.float32),
        pltpu.VMEM((CH, D), jnp.float32),
        pltpu.VMEM_SHARED((NPAD, D), jnp.float32),
        pltpu.SemaphoreType.DMA,
    ],
    compiler_params=_SC_PARAMS_NT,
)
def _agg_kernel(table_hbm, src_hbm, dst_hbm, zeros_hbm, out_hbm,
                src_v, dst_v, rpk_v, rf_v, acc_sh, sem):
    c = lax.axis_index("c")
    s = lax.axis_index("s")
    wid = c * NS + s

    @pl.when(s == 0)
    def _():
        pltpu.sync_copy(zeros_hbm, acc_sh)

    pltpu.sync_copy(src_hbm.at[wid], src_v)
    pltpu.sync_copy(dst_hbm.at[wid], dst_v)
    plsc.subcore_barrier()

    def chunk(k, carry):
        pltpu.async_copy(table_hbm.at[src_v.at[k]], rpk_v, sem).wait()
        for r in range(CH):
            for j in range(D // 32):
                pk = plsc.bitcast(rpk_v[r, pl.ds(j * L, L)], jnp.bfloat16)
                lo, hi = plsc.unpack(pk, format=plsc.PackFormat.INTERLEAVED)
                rf_v[r, pl.ds(j * L, L)] = lo
                rf_v[r, pl.ds(D // 2 + j * L, L)] = hi
        pltpu.sync_copy(rf_v, acc_sh.at[dst_v.at[k]], add=True)
        return carry

    lax.fori_loop(0, K_AGG, chunk, 0)
    plsc.subcore_barrier()
    pltpu.sync_copy(
        acc_sh.at[pl.ds(s * ROWS_PER_TILE, ROWS_PER_TILE)],
        out_hbm.at[pl.ds(c * NPAD + s * ROWS_PER_TILE, ROWS_PER_TILE)])


@functools.partial(
    pl.kernel,
    out_type=jax.ShapeDtypeStruct((NW * K_DEC, CH), jnp.float32),
    mesh=_MESH,
    scratch_types=[
        pltpu.VMEM((K_DEC, CH), jnp.int32),
        pltpu.VMEM((K_DEC, CH), jnp.int32),
        pltpu.VMEM((4, CH, D // 2), jnp.float32),
        pltpu.VMEM((4, CH, D // 2), jnp.float32),
        pltpu.VMEM((L, L), jnp.float32),
        pltpu.VMEM((K_DEC, CH), jnp.float32),
        pltpu.SemaphoreType.DMA((4,)),
    ],
    compiler_params=_SC_PARAMS_NT,
)
def _dec_kernel(z_hbm, ia_hbm, ib_hbm, out_hbm, ia_v, ib_v, ra_v, rb_v, pt_v,
                sc_v, sems):
    c = lax.axis_index("c")
    s = lax.axis_index("s")
    wid = c * NS + s
    pltpu.sync_copy(ia_hbm.at[wid], ia_v)
    pltpu.sync_copy(ib_hbm.at[wid], ib_v)
    lane = lax.iota(jnp.int32, 16)

    def _issue(k, b):
        pltpu.async_copy(z_hbm.at[ia_v.at[k]], ra_v.at[b], sems.at[b])
        pltpu.async_copy(z_hbm.at[ib_v.at[k]], rb_v.at[b], sems.at[b])

    def _drain(k, b):
        pltpu.make_async_copy(z_hbm.at[ia_v.at[k]], ra_v.at[b], sems.at[b]).wait()
        pltpu.make_async_copy(z_hbm.at[ib_v.at[k]], rb_v.at[b], sems.at[b]).wait()

    def _compute(k, b):
        ra = ra_v.at[b]
        rb = rb_v.at[b]

        def group(g, carry2):
            for e in range(L):
                row = g * L + e
                ps = []
                for j in range(D // 32):
                    apk = plsc.bitcast(ra[row, pl.ds(j * L, L)], jnp.bfloat16)
                    bpk = plsc.bitcast(rb[row, pl.ds(j * L, L)], jnp.bfloat16)
                    a0, a1 = plsc.unpack(apk, format=plsc.PackFormat.INTERLEAVED)
                    b0, b1 = plsc.unpack(bpk, format=plsc.PackFormat.INTERLEAVED)
                    ps.append(a0 * b0)
                    ps.append(a1 * b1)
                part = (((ps[0] + ps[1]) + (ps[2] + ps[3]))
                        + ((ps[4] + ps[5]) + (ps[6] + ps[7])))
                pt_v[e] = part
            score = plsc.load_gather(pt_v, [lane, jnp.zeros((L,), jnp.int32)])
            for j in range(1, L):
                score = score + plsc.load_gather(
                    pt_v, [lane, jnp.full((L,), j, jnp.int32)])
            sc_v[k, pl.ds(g * L, L)] = score
            return carry2

        lax.fori_loop(0, CH // L, group, 0)

    for b in range(4):
        _issue(b, b)

    def ring(g, carry):
        for b in range(4):
            k = 4 * g + b
            _drain(k, b)
            _compute(k, b)

            @pl.when(k + 4 < K_DEC)
            def _():
                _issue(k + 4, b)
        return carry

    lax.fori_loop(0, K_DEC // 4, ring, 0)
    pltpu.sync_copy(sc_v, out_hbm.at[pl.ds(wid * K_DEC, K_DEC)])


_R = 1280


def _dinv_of(d_ref):
    deg = jnp.sum(d_ref[...], axis=0)[:, None]
    return jnp.where(deg > 0, lax.rsqrt(deg), 0.0)


def _tc1_body(x_ref, w_ref, d_ref, o_ref):
    dinv = _dinv_of(d_ref)
    xw = jnp.dot(x_ref[...], w_ref[...],
                 preferred_element_type=jnp.float32) * dinv
    o_ref[...] = xw.astype(jnp.bfloat16)


def _tc2_body(p0_ref, p1_ref, d_ref, b_ref, w_ref, o_ref):
    dinv = _dinv_of(d_ref)
    h = jnp.maximum((p0_ref[...] + p1_ref[...]) * dinv + b_ref[...], 0.0)
    xw = jnp.dot(h, w_ref[...], preferred_element_type=jnp.float32) * dinv
    o_ref[...] = xw.astype(jnp.bfloat16)


def _tc3_body(p0_ref, p1_ref, d_ref, b_ref, o_ref, ob_ref):
    dinv = _dinv_of(d_ref)
    z = (p0_ref[...] + p1_ref[...]) * dinv + b_ref[...]
    o_ref[...] = z
    ob_ref[...] = z.astype(jnp.bfloat16)


def _row_spec(cols):
    return pl.BlockSpec((_R, cols), lambda i: (i, 0))


def _full_spec(rows, cols):
    return pl.BlockSpec((rows, cols), lambda i: (0, 0))


_deg_spec = pl.BlockSpec((NW, _R), lambda i: (0, i))

_tc1 = pl.pallas_call(
    _tc1_body,
    grid=(NPAD // _R,),
    in_specs=[_row_spec(D), _full_spec(D, D), _deg_spec],
    out_specs=_row_spec(D),
    out_shape=jax.ShapeDtypeStruct((NPAD, D), jnp.bfloat16),
)

_tc2 = pl.pallas_call(
    _tc2_body,
    grid=(NPAD // _R,),
    in_specs=[_row_spec(D), _row_spec(D), _deg_spec,
              _full_spec(1, D), _full_spec(D, D)],
    out_specs=_row_spec(D),
    out_shape=jax.ShapeDtypeStruct((NPAD, D), jnp.bfloat16),
)


def _pack_cols(b16):
    pairs = jnp.stack([b16[:, :D // 2], b16[:, D // 2:]], axis=-1)
    return jax.lax.bitcast_convert_type(pairs, jnp.float32)

_tc3 = pl.pallas_call(
    _tc3_body,
    grid=(NPAD // _R,),
    in_specs=[_row_spec(D), _row_spec(D), _deg_spec, _full_spec(1, D)],
    out_specs=[_row_spec(D), _row_spec(D)],
    out_shape=[jax.ShapeDtypeStruct((NPAD, D), jnp.float32),
               jax.ShapeDtypeStruct((NPAD, D), jnp.bfloat16)],
)


def kernel(x, edge_index, pos_edge_index, neg_edge_index, W1, b1, W2, b2):
    f32 = jnp.float32
    i32 = jnp.int32
    n_edges = edge_index.shape[1]

    src = edge_index[0].astype(i32)
    dst = edge_index[1].astype(i32)
    loop = jnp.arange(N_NODES, dtype=i32)
    pad_a = jnp.full((E_AGG - n_edges - N_NODES,), DUMMY, i32)
    src_sl = jnp.concatenate([src, loop, pad_a]).reshape(NW, K_AGG, CH)
    dst_sl = jnp.concatenate([dst, loop, pad_a]).reshape(NW, K_AGG, CH)

    zeros_d = jnp.zeros((NPAD, D), f32)
    x_pad = jnp.concatenate([x, jnp.zeros((NPAD - N_NODES, D), f32)], axis=0)

    degs = _deg_kernel(dst_sl)

    xw1 = _tc1(x_pad, W1, degs)
    acc1 = _agg_kernel(_pack_cols(xw1), src_sl, dst_sl, zeros_d)
    xw2 = _tc2(acc1[:NPAD], acc1[NPAD:], degs, b1.reshape(1, D), W2)
    acc2 = _agg_kernel(_pack_cols(xw2), src_sl, dst_sl, zeros_d)
    z_pad, z_b16 = _tc3(acc2[:NPAD], acc2[NPAD:], degs, b2.reshape(1, D))
    z_pk = jax.lax.bitcast_convert_type(z_b16.reshape(NPAD, D // 2, 2),
                                        jnp.float32)

    n_dec = pos_edge_index.shape[1]
    pad_d = jnp.full((E_DEC - n_dec,), DUMMY, i32)

    def _dec_idx(arr):
        return jnp.concatenate([arr.astype(i32), pad_d]).reshape(NW, K_DEC, CH)

    pos = _dec_kernel(z_pk, _dec_idx(pos_edge_index[0]),
                      _dec_idx(pos_edge_index[1])).reshape(-1)[:n_dec]
    neg = _dec_kernel(z_pk, _dec_idx(neg_edge_index[0]),
                      _dec_idx(neg_edge_index[1])).reshape(-1)[:n_dec]
    return pos, neg, z_pad[:N_NODES]
